# Initial kernel scaffold; baseline (speedup 1.0000x reference)
import functools
import math

import jax
import jax.numpy as jnp
from jax import lax
from jax.experimental import pallas as pl
from jax.experimental.pallas import tpu as pltpu
from jax.experimental.pallas import tpu_sc as plsc

_KNN = 5
_S = 5
_SS = _S * _S
_CENTER = (_SS - 1) // 2
_SIGMA = 1.0
_CUTOFF = 1.0
_NCLS = 20
_KROWS = 32
_UNP_ROW = 25

_NC, _NS = 2, 16
_NW = _NC * _NS


def _inv_gauss_weights():
    x = jnp.arange(_S)
    x_grid = jnp.tile(x, _S).reshape(_S, _S)
    y_grid = x_grid.T
    mean = (_S - 1) / 2.0
    var = _SIGMA ** 2.0
    g = (1.0 / (2.0 * math.pi * var)) * jnp.exp(
        -((x_grid - mean) ** 2.0 + (y_grid - mean) ** 2.0) / (2.0 * var))
    g = g / jnp.sum(g)
    w = (1.0 - g).reshape(_SS).astype(jnp.float32)
    return jnp.concatenate([w, jnp.zeros((_KROWS - _SS,), jnp.float32)])


def _sc_gather(idx_all, rng_pad, cls_pad, unproj, n_points):
    pt = n_points // _NW
    mesh = plsc.VectorSubcoreMesh(core_axis_name="c", subcore_axis_name="s",
                                  num_cores=_NC, num_subcores=_NS)

    @functools.partial(
        pl.kernel,
        out_type=(jax.ShapeDtypeStruct((_KROWS, n_points), jnp.float32),
                  jax.ShapeDtypeStruct((_KROWS, n_points), jnp.int32)),
        mesh=mesh,
        scratch_types=[
            pltpu.VMEM((pt,), jnp.int32),
            pltpu.VMEM((pt,), jnp.float32),
            pltpu.VMEM((pt,), jnp.int32),
            pltpu.VMEM((pt,), jnp.float32),
            pltpu.SemaphoreType.DMA,
        ],
    )
    def sc_kernel(idx_hbm, rng_hbm, cls_hbm, unp_hbm, grng_hbm, gcls_hbm,
                  idx_v, bufr_v, bufc_v, unp_v, sem):
        wid = lax.axis_index("s") * _NC + lax.axis_index("c")
        base = wid * pt
        pltpu.sync_copy(unp_hbm.at[pl.ds(base, pt)], unp_v)
        pltpu.sync_copy(unp_v, grng_hbm.at[_CENTER, pl.ds(base, pt)])
        pltpu.sync_copy(unp_v, grng_hbm.at[_UNP_ROW, pl.ds(base, pt)])
        for k in range(_SS):
            pltpu.sync_copy(idx_hbm.at[k, pl.ds(base, pt)], idx_v)
            if k != _CENTER:
                pltpu.async_copy(rng_hbm.at[idx_v], bufr_v, sem).wait()
                pltpu.sync_copy(bufr_v, grng_hbm.at[k, pl.ds(base, pt)])
            pltpu.async_copy(cls_hbm.at[idx_v], bufc_v, sem).wait()
            pltpu.sync_copy(bufc_v, gcls_hbm.at[k, pl.ds(base, pt)])

    return sc_kernel(idx_all, rng_pad, cls_pad, unproj)


def _tc_body(grng_ref, gcls_ref, w_ref, o_ref):
    g = grng_ref[...]
    cls_ = gcls_ref[...]
    w = w_ref[...]
    b = g.shape[1]
    r = g[_UNP_ROW:_UNP_ROW + 1, :]
    rows = lax.broadcasted_iota(jnp.int32, (_KROWS, b), 0)
    d = jnp.abs(g - r) * w
    d = jnp.where(rows < _SS, d, jnp.inf)

    sel_cls = []
    for _ in range(_KNN):
        m = jnp.min(d, axis=0, keepdims=True)
        ism = d == m
        ki = jnp.min(jnp.where(ism, rows, _KROWS), axis=0, keepdims=True)
        hit = rows == ki
        c = jnp.max(jnp.where(hit, cls_, -1), axis=0, keepdims=True)
        c = jnp.where(m > _CUTOFF, _NCLS, c)
        sel_cls.append(c)
        d = jnp.where(hit, jnp.inf, d)

    best_cnt = sum((c == 1).astype(jnp.int32) for c in sel_cls)
    best_cls = jnp.ones_like(best_cnt)
    for cc in range(2, _NCLS):
        cnt = sum((c == cc).astype(jnp.int32) for c in sel_cls)
        upd = cnt > best_cnt
        best_cnt = jnp.where(upd, cnt, best_cnt)
        best_cls = jnp.where(upd, cc, best_cls)
    o_ref[...] = best_cls.reshape(1, 1, b)


def _tc_compute(g_rng, g_cls, w_col, n_points, block=2048):
    nb = n_points // block
    return pl.pallas_call(
        _tc_body,
        grid=(nb,),
        in_specs=[
            pl.BlockSpec((_KROWS, block), lambda i: (0, i)),
            pl.BlockSpec((_KROWS, block), lambda i: (0, i)),
            pl.BlockSpec((_KROWS, 1), lambda i: (0, 0)),
        ],
        out_specs=pl.BlockSpec((1, 1, block), lambda i: (i, 0, 0)),
        out_shape=jax.ShapeDtypeStruct((nb, 1, block), jnp.int32),
    )(g_rng, g_cls, w_col)


def kernel(proj_range, unproj_range, proj_argmax, px, py):
    h, w = proj_range.shape
    p = unproj_range.shape[0]
    pad = (_S - 1) // 2
    wp = w + 2 * pad
    rng_pad = jnp.pad(proj_range, pad).reshape(-1)
    cls_pad = jnp.pad(proj_argmax, pad).reshape(-1)
    base = py * wp + px
    offs = jnp.array([dy * wp + dx for dy in range(_S) for dx in range(_S)],
                     jnp.int32)
    idx_all = offs[:, None] + base[None, :]
    g_rng, g_cls = _sc_gather(idx_all, rng_pad, cls_pad, unproj_range, p)
    w_col = _inv_gauss_weights().reshape(_KROWS, 1)
    out3 = _tc_compute(g_rng, g_cls, w_col, p)
    return out3.reshape(p)


# speedup vs baseline: 5.6825x; 5.6825x over previous
import functools
import math

import jax
import jax.numpy as jnp
from jax import lax
from jax.experimental import pallas as pl
from jax.experimental.pallas import tpu as pltpu
from jax.experimental.pallas import tpu_sc as plsc

_KNN = 5
_S = 5
_SS = _S * _S
_CENTER = (_SS - 1) // 2
_SIGMA = 1.0
_CUTOFF = 1.0
_NCLS = 20
_KROWS = 32
_UNP_ROW = 25

_NC, _NS = 2, 16
_NW = _NC * _NS


def _inv_gauss_weights():
    x = jnp.arange(_S)
    x_grid = jnp.tile(x, _S).reshape(_S, _S)
    y_grid = x_grid.T
    mean = (_S - 1) / 2.0
    var = _SIGMA ** 2.0
    g = (1.0 / (2.0 * math.pi * var)) * jnp.exp(
        -((x_grid - mean) ** 2.0 + (y_grid - mean) ** 2.0) / (2.0 * var))
    g = g / jnp.sum(g)
    w = (1.0 - g).reshape(_SS).astype(jnp.float32)
    return jnp.concatenate([w, jnp.zeros((_KROWS - _SS,), jnp.float32)])


def _sc_gather(idx_all, rng_pad, cls_pad, unproj, n_points):
    pt = n_points // _NW
    mesh = plsc.VectorSubcoreMesh(core_axis_name="c", subcore_axis_name="s",
                                  num_cores=_NC, num_subcores=_NS)

    @functools.partial(
        pl.kernel,
        out_type=(jax.ShapeDtypeStruct((_KROWS * n_points,), jnp.float32),
                  jax.ShapeDtypeStruct((_KROWS * n_points,), jnp.int32)),
        mesh=mesh,
        scratch_types=[
            pltpu.VMEM((pt,), jnp.int32),
            pltpu.VMEM((pt,), jnp.float32),
            pltpu.VMEM((pt,), jnp.int32),
            pltpu.VMEM((pt,), jnp.float32),
            pltpu.SemaphoreType.DMA,
        ],
    )
    def sc_kernel(idx_hbm, rng_hbm, cls_hbm, unp_hbm, grng_hbm, gcls_hbm,
                  idx_v, bufr_v, bufc_v, unp_v, sem):
        wid = lax.axis_index("s") * _NC + lax.axis_index("c")
        base = wid * pt
        pltpu.sync_copy(unp_hbm.at[pl.ds(base, pt)], unp_v)
        pltpu.sync_copy(unp_v, grng_hbm.at[pl.ds(_CENTER * n_points + base, pt)])
        pltpu.sync_copy(unp_v, grng_hbm.at[pl.ds(_UNP_ROW * n_points + base, pt)])
        for k in range(_SS):
            pltpu.sync_copy(idx_hbm.at[pl.ds(k * n_points + base, pt)], idx_v)
            if k != _CENTER:
                pltpu.async_copy(rng_hbm.at[idx_v], bufr_v, sem).wait()
                pltpu.sync_copy(
                    bufr_v, grng_hbm.at[pl.ds(k * n_points + base, pt)])
            pltpu.async_copy(cls_hbm.at[idx_v], bufc_v, sem).wait()
            pltpu.sync_copy(
                bufc_v, gcls_hbm.at[pl.ds(k * n_points + base, pt)])

    g_rng, g_cls = sc_kernel(idx_all.reshape(-1), rng_pad, cls_pad, unproj)
    return (g_rng.reshape(_KROWS, n_points), g_cls.reshape(_KROWS, n_points))


def _tc_body(grng_ref, gcls_ref, w_ref, o_ref):
    g = grng_ref[...]
    cls_ = gcls_ref[...]
    w = w_ref[...]
    b = g.shape[1]
    r = g[_UNP_ROW:_UNP_ROW + 1, :]
    rows = lax.broadcasted_iota(jnp.int32, (_KROWS, b), 0)
    d = jnp.abs(g - r) * w
    d = jnp.where(rows < _SS, d, jnp.inf)

    sel_cls = []
    for _ in range(_KNN):
        m = jnp.min(d, axis=0, keepdims=True)
        ism = d == m
        ki = jnp.min(jnp.where(ism, rows, _KROWS), axis=0, keepdims=True)
        hit = rows == ki
        c = jnp.max(jnp.where(hit, cls_, -1), axis=0, keepdims=True)
        c = jnp.where(m > _CUTOFF, _NCLS, c)
        sel_cls.append(c)
        d = jnp.where(hit, jnp.inf, d)

    best_cnt = sum((c == 1).astype(jnp.int32) for c in sel_cls)
    best_cls = jnp.ones_like(best_cnt)
    for cc in range(2, _NCLS):
        cnt = sum((c == cc).astype(jnp.int32) for c in sel_cls)
        upd = cnt > best_cnt
        best_cnt = jnp.where(upd, cnt, best_cnt)
        best_cls = jnp.where(upd, cc, best_cls)
    o_ref[...] = best_cls.reshape(1, 1, b)


def _tc_compute(g_rng, g_cls, w_col, n_points, block=2048):
    nb = n_points // block
    return pl.pallas_call(
        _tc_body,
        grid=(nb,),
        in_specs=[
            pl.BlockSpec((_KROWS, block), lambda i: (0, i)),
            pl.BlockSpec((_KROWS, block), lambda i: (0, i)),
            pl.BlockSpec((_KROWS, 1), lambda i: (0, 0)),
        ],
        out_specs=pl.BlockSpec((1, 1, block), lambda i: (i, 0, 0)),
        out_shape=jax.ShapeDtypeStruct((nb, 1, block), jnp.int32),
    )(g_rng, g_cls, w_col)


def kernel(proj_range, unproj_range, proj_argmax, px, py):
    h, w = proj_range.shape
    p = unproj_range.shape[0]
    pad = (_S - 1) // 2
    wp = w + 2 * pad
    rng_pad = jnp.pad(proj_range, pad).reshape(-1)
    cls_pad = jnp.pad(proj_argmax, pad).reshape(-1)
    base = py * wp + px
    offs = jnp.array([dy * wp + dx for dy in range(_S) for dx in range(_S)],
                     jnp.int32)
    idx_all = offs[:, None] + base[None, :]
    g_rng, g_cls = _sc_gather(idx_all, rng_pad, cls_pad, unproj_range, p)
    w_col = _inv_gauss_weights().reshape(_KROWS, 1)
    out3 = _tc_compute(g_rng, g_cls, w_col, p)
    return out3.reshape(p)


# speedup vs baseline: 6.5291x; 1.1490x over previous
import functools
import math

import jax
import jax.numpy as jnp
from jax import lax
from jax.experimental import pallas as pl
from jax.experimental.pallas import tpu as pltpu
from jax.experimental.pallas import tpu_sc as plsc

_KNN = 5
_S = 5
_SS = _S * _S
_CENTER = (_SS - 1) // 2
_SIGMA = 1.0
_CUTOFF = 1.0
_NCLS = 20
_KROWS = 32
_UNP_ROW = 25

_NC, _NS = 2, 16
_NW = _NC * _NS


def _inv_gauss_weights():
    x = jnp.arange(_S)
    x_grid = jnp.tile(x, _S).reshape(_S, _S)
    y_grid = x_grid.T
    mean = (_S - 1) / 2.0
    var = _SIGMA ** 2.0
    g = (1.0 / (2.0 * math.pi * var)) * jnp.exp(
        -((x_grid - mean) ** 2.0 + (y_grid - mean) ** 2.0) / (2.0 * var))
    g = g / jnp.sum(g)
    w = (1.0 - g).reshape(_SS).astype(jnp.float32)
    return jnp.concatenate([w, jnp.zeros((_KROWS - _SS,), jnp.float32)])


def _sc_gather(idx_all, rng_pad, cls_pad, unproj, n_points):
    pt = n_points // _NW
    mesh = plsc.VectorSubcoreMesh(core_axis_name="c", subcore_axis_name="s",
                                  num_cores=_NC, num_subcores=_NS)

    nbuf = 3
    scratch = ([pltpu.VMEM((pt,), jnp.int32) for _ in range(nbuf)]
               + [pltpu.VMEM((pt,), jnp.float32) for _ in range(nbuf)]
               + [pltpu.VMEM((pt,), jnp.int32) for _ in range(nbuf)]
               + [pltpu.VMEM((pt,), jnp.float32)]
               + [pltpu.SemaphoreType.DMA for _ in range(3 * nbuf + 1)])

    @functools.partial(
        pl.kernel,
        out_type=(jax.ShapeDtypeStruct((_KROWS * n_points,), jnp.float32),
                  jax.ShapeDtypeStruct((_KROWS * n_points,), jnp.int32)),
        mesh=mesh,
        scratch_types=scratch,
    )
    def sc_kernel(idx_hbm, rng_hbm, cls_hbm, unp_hbm, grng_hbm, gcls_hbm,
                  *bufs):
        idx_v = bufs[0:nbuf]
        bufr_v = bufs[nbuf:2 * nbuf]
        bufc_v = bufs[2 * nbuf:3 * nbuf]
        unp_v = bufs[3 * nbuf]
        semi = bufs[3 * nbuf + 1:3 * nbuf + 1 + nbuf]
        semg = bufs[3 * nbuf + 1 + nbuf:3 * nbuf + 1 + 2 * nbuf]
        sems = bufs[3 * nbuf + 1 + 2 * nbuf:3 * nbuf + 1 + 3 * nbuf]
        semu = bufs[3 * nbuf + 1 + 3 * nbuf]
        wid = lax.axis_index("s") * _NC + lax.axis_index("c")
        base = wid * pt

        def outrow(k):
            return pl.ds(k * n_points + base, pt)

        pltpu.sync_copy(unp_hbm.at[pl.ds(base, pt)], unp_v)
        u1 = pltpu.async_copy(unp_v, grng_hbm.at[outrow(_CENTER)], semu)
        u2 = pltpu.async_copy(unp_v, grng_hbm.at[outrow(_UNP_ROW)], semu)

        ld = [None] * _SS
        gat = [None] * _SS
        st = [None] * _SS

        def fire_gat(k):
            s = k % nbuf
            g = []
            if k != _CENTER:
                g.append(
                    pltpu.async_copy(rng_hbm.at[idx_v[s]], bufr_v[s], semg[s]))
            g.append(
                pltpu.async_copy(cls_hbm.at[idx_v[s]], bufc_v[s], semg[s]))
            gat[k] = g

        def fire_st(k):
            s = k % nbuf
            for h in gat[k]:
                h.wait()
            g = []
            if k != _CENTER:
                g.append(
                    pltpu.async_copy(bufr_v[s], grng_hbm.at[outrow(k)], sems[s]))
            g.append(
                pltpu.async_copy(bufc_v[s], gcls_hbm.at[outrow(k)], sems[s]))
            st[k] = g

        ld[0] = pltpu.async_copy(idx_hbm.at[outrow(0)], idx_v[0], semi[0])
        for k in range(_SS):
            s = k % nbuf
            ld[k].wait()
            if k >= nbuf:
                for h in st[k - nbuf]:
                    h.wait()
            fire_gat(k)
            if k + 1 < _SS:
                if k + 1 >= nbuf:
                    fire_st(k + 1 - nbuf)
                ld[k + 1] = pltpu.async_copy(
                    idx_hbm.at[outrow(k + 1)], idx_v[(k + 1) % nbuf],
                    semi[(k + 1) % nbuf])
        for k in range(_SS - nbuf, _SS):
            fire_st(k)
            for h in st[k]:
                h.wait()
        u1.wait()
        u2.wait()

    g_rng, g_cls = sc_kernel(idx_all.reshape(-1), rng_pad, cls_pad, unproj)
    return (g_rng.reshape(_KROWS, n_points), g_cls.reshape(_KROWS, n_points))


def _tc_body(grng_ref, gcls_ref, w_ref, o_ref):
    g = grng_ref[...]
    cls_ = gcls_ref[...]
    w = w_ref[...]
    b = g.shape[1]
    r = g[_UNP_ROW:_UNP_ROW + 1, :]
    rows = lax.broadcasted_iota(jnp.int32, (_KROWS, b), 0)
    d = jnp.abs(g - r) * w
    d = jnp.where(rows < _SS, d, jnp.inf)

    sel_cls = []
    for _ in range(_KNN):
        m = jnp.min(d, axis=0, keepdims=True)
        ism = d == m
        ki = jnp.min(jnp.where(ism, rows, _KROWS), axis=0, keepdims=True)
        hit = rows == ki
        c = jnp.max(jnp.where(hit, cls_, -1), axis=0, keepdims=True)
        c = jnp.where(m > _CUTOFF, _NCLS, c)
        sel_cls.append(c)
        d = jnp.where(hit, jnp.inf, d)

    ones = jnp.ones_like(sel_cls[0], dtype=jnp.int32)
    cnt = [ones, ones, ones, ones, ones]
    for i in range(_KNN):
        for j in range(i + 1, _KNN):
            e = (sel_cls[i] == sel_cls[j]).astype(jnp.int32)
            cnt[i] = cnt[i] + e
            cnt[j] = cnt[j] + e
    neg = jnp.full_like(ones, -1000)
    key = neg
    for i in range(_KNN):
        c = sel_cls[i]
        valid = (c >= 1) & (c < _NCLS)
        key = jnp.maximum(key, jnp.where(valid, cnt[i] * 32 - c, neg))
    best_cls = jnp.where(key == -1000, 1, 32 - (key & 31))
    o_ref[...] = best_cls.reshape(1, 1, b)


def _tc_compute(g_rng, g_cls, w_col, n_points, block=2048):
    nb = n_points // block
    return pl.pallas_call(
        _tc_body,
        grid=(nb,),
        in_specs=[
            pl.BlockSpec((_KROWS, block), lambda i: (0, i)),
            pl.BlockSpec((_KROWS, block), lambda i: (0, i)),
            pl.BlockSpec((_KROWS, 1), lambda i: (0, 0)),
        ],
        out_specs=pl.BlockSpec((1, 1, block), lambda i: (i, 0, 0)),
        out_shape=jax.ShapeDtypeStruct((nb, 1, block), jnp.int32),
    )(g_rng, g_cls, w_col)


def kernel(proj_range, unproj_range, proj_argmax, px, py):
    h, w = proj_range.shape
    p = unproj_range.shape[0]
    pad = (_S - 1) // 2
    wp = w + 2 * pad
    rng_pad = jnp.pad(proj_range, pad).reshape(-1)
    cls_pad = jnp.pad(proj_argmax, pad).reshape(-1)
    base = py * wp + px
    offs = jnp.array([dy * wp + dx for dy in range(_S) for dx in range(_S)],
                     jnp.int32)
    idx_all = offs[:, None] + base[None, :]
    g_rng, g_cls = _sc_gather(idx_all, rng_pad, cls_pad, unproj_range, p)
    w_col = _inv_gauss_weights().reshape(_KROWS, 1)
    out3 = _tc_compute(g_rng, g_cls, w_col, p)
    return out3.reshape(p)


# speedup vs baseline: 7.6907x; 1.1779x over previous
import functools
import math

import jax
import jax.numpy as jnp
from jax import lax
from jax.experimental import pallas as pl
from jax.experimental.pallas import tpu as pltpu
from jax.experimental.pallas import tpu_sc as plsc

_KNN = 5
_S = 5
_SS = _S * _S
_CENTER = (_SS - 1) // 2
_SIGMA = 1.0
_CUTOFF = 1.0
_NCLS = 20
_KROWS = 32
_UNP_ROW = 25
_SROWS = 8

_NC, _NS = 2, 16
_NW = _NC * _NS
_NBUF = 3


def _inv_gauss_weights():
    x = jnp.arange(_S)
    x_grid = jnp.tile(x, _S).reshape(_S, _S)
    y_grid = x_grid.T
    mean = (_S - 1) / 2.0
    var = _SIGMA ** 2.0
    g = (1.0 / (2.0 * math.pi * var)) * jnp.exp(
        -((x_grid - mean) ** 2.0 + (y_grid - mean) ** 2.0) / (2.0 * var))
    g = g / jnp.sum(g)
    w = (1.0 - g).reshape(_SS).astype(jnp.float32)
    return jnp.concatenate([w, jnp.zeros((_KROWS - _SS,), jnp.float32)])


def _pipelined_gather(table_hbm, idx_hbm, out_hbm, bufs, ks, n_points, pt,
                      base):
    idx_v = bufs[0:_NBUF]
    buf_v = bufs[_NBUF:2 * _NBUF]
    semi = bufs[2 * _NBUF:3 * _NBUF]
    semg = bufs[3 * _NBUF:4 * _NBUF]
    sems = bufs[4 * _NBUF:5 * _NBUF]

    def row(k):
        return pl.ds(k * n_points + base, pt)

    nk = len(ks)
    ld = [None] * nk
    gat = [None] * nk
    st = [None] * nk

    def fire_st(i):
        s = i % _NBUF
        gat[i].wait()
        st[i] = pltpu.async_copy(buf_v[s], out_hbm.at[row(ks[i])], sems[s])

    ld[0] = pltpu.async_copy(idx_hbm.at[row(ks[0])], idx_v[0], semi[0])
    for i in range(nk):
        s = i % _NBUF
        ld[i].wait()
        if i >= _NBUF:
            st[i - _NBUF].wait()
        gat[i] = pltpu.async_copy(table_hbm.at[idx_v[s]], buf_v[s], semg[s])
        if i + 1 < nk:
            if i + 1 >= _NBUF:
                fire_st(i + 1 - _NBUF)
            ld[i + 1] = pltpu.async_copy(
                idx_hbm.at[row(ks[i + 1])], idx_v[(i + 1) % _NBUF],
                semi[(i + 1) % _NBUF])
    for i in range(max(0, nk - _NBUF), nk):
        fire_st(i)
        st[i].wait()


def _sc_gather_range(idx_all, rng_pad, unproj, n_points):
    pt = n_points // _NW
    mesh = plsc.VectorSubcoreMesh(core_axis_name="c", subcore_axis_name="s",
                                  num_cores=_NC, num_subcores=_NS)
    scratch = ([pltpu.VMEM((pt,), jnp.int32) for _ in range(_NBUF)]
               + [pltpu.VMEM((pt,), jnp.float32) for _ in range(_NBUF)]
               + [pltpu.SemaphoreType.DMA for _ in range(3 * _NBUF)]
               + [pltpu.VMEM((pt,), jnp.float32), pltpu.SemaphoreType.DMA])

    @functools.partial(
        pl.kernel,
        out_type=jax.ShapeDtypeStruct((_KROWS * n_points,), jnp.float32),
        mesh=mesh,
        scratch_types=scratch,
    )
    def sc1(idx_hbm, rng_hbm, unp_hbm, grng_hbm, *bufs):
        unp_v = bufs[5 * _NBUF]
        semu = bufs[5 * _NBUF + 1]
        wid = lax.axis_index("s") * _NC + lax.axis_index("c")
        base = wid * pt
        pltpu.sync_copy(unp_hbm.at[pl.ds(base, pt)], unp_v)
        u1 = pltpu.async_copy(
            unp_v, grng_hbm.at[pl.ds(_CENTER * n_points + base, pt)], semu)
        u2 = pltpu.async_copy(
            unp_v, grng_hbm.at[pl.ds(_UNP_ROW * n_points + base, pt)], semu)
        ks = [k for k in range(_SS) if k != _CENTER]
        _pipelined_gather(rng_hbm, idx_hbm, grng_hbm, bufs[:5 * _NBUF], ks,
                          n_points, pt, base)
        u1.wait()
        u2.wait()

    return sc1(idx_all, rng_pad, unproj)


def _sc_gather_cls(sel_idx, cls_pad, n_points):
    pt = n_points // _NW
    mesh = plsc.VectorSubcoreMesh(core_axis_name="c", subcore_axis_name="s",
                                  num_cores=_NC, num_subcores=_NS)
    scratch = ([pltpu.VMEM((pt,), jnp.int32) for _ in range(_NBUF)]
               + [pltpu.VMEM((pt,), jnp.int32) for _ in range(_NBUF)]
               + [pltpu.SemaphoreType.DMA for _ in range(3 * _NBUF)])

    @functools.partial(
        pl.kernel,
        out_type=jax.ShapeDtypeStruct((_SROWS * n_points,), jnp.int32),
        mesh=mesh,
        scratch_types=scratch,
    )
    def sc2(selidx_hbm, cls_hbm, cls5_hbm, *bufs):
        wid = lax.axis_index("s") * _NC + lax.axis_index("c")
        base = wid * pt
        _pipelined_gather(cls_hbm, selidx_hbm, cls5_hbm, bufs,
                          list(range(_KNN)), n_points, pt, base)

    return sc2(sel_idx, cls_pad)


def _tc_select_body(sentinel, grng_ref, base_ref, w_ref, offs_ref, o_ref):
    g = grng_ref[...]
    w = w_ref[...]
    offs = offs_ref[...]
    b = g.shape[1]
    base = base_ref[...].reshape(1, b)
    r = g[_UNP_ROW:_UNP_ROW + 1, :]
    rows = lax.broadcasted_iota(jnp.int32, (_KROWS, b), 0)
    d = jnp.abs(g - r) * w
    d = jnp.where(rows < _SS, d, jnp.inf)

    sel = []
    for _ in range(_KNN):
        m = jnp.min(d, axis=0, keepdims=True)
        ki = jnp.min(jnp.where(d == m, rows, _KROWS), axis=0, keepdims=True)
        hit = rows == ki
        off = jnp.max(jnp.where(hit, offs, -1), axis=0, keepdims=True)
        flat = jnp.where(m > _CUTOFF, sentinel, base + off)
        sel.append(flat)
        d = jnp.where(hit, jnp.inf, d)
    zero = jnp.zeros_like(sel[0])
    o_ref[...] = jnp.concatenate(sel + [zero] * (_SROWS - _KNN), axis=0)


def _tc_select(g_rng, base3, w_col, offs_col, sentinel, n_points, block=2048):
    nb = n_points // block
    return pl.pallas_call(
        functools.partial(_tc_select_body, sentinel),
        grid=(nb,),
        in_specs=[
            pl.BlockSpec((_KROWS, block), lambda i: (0, i)),
            pl.BlockSpec((1, 1, block), lambda i: (i, 0, 0)),
            pl.BlockSpec((_KROWS, 1), lambda i: (0, 0)),
            pl.BlockSpec((_KROWS, 1), lambda i: (0, 0)),
        ],
        out_specs=pl.BlockSpec((_SROWS, block), lambda i: (0, i)),
        out_shape=jax.ShapeDtypeStruct((_SROWS, n_points), jnp.int32),
    )(g_rng, base3, w_col, offs_col)


def _tc_vote_body(cls_ref, o_ref):
    cl = cls_ref[...]
    b = cl.shape[1]
    sel = [cl[i:i + 1, :] for i in range(_KNN)]
    ones = jnp.ones_like(sel[0])
    cnt = [ones] * _KNN
    for i in range(_KNN):
        for j in range(i + 1, _KNN):
            e = (sel[i] == sel[j]).astype(jnp.int32)
            cnt[i] = cnt[i] + e
            cnt[j] = cnt[j] + e
    neg = jnp.full_like(ones, -1000)
    key = neg
    for i in range(_KNN):
        c = sel[i]
        valid = (c >= 1) & (c < _NCLS)
        key = jnp.maximum(key, jnp.where(valid, cnt[i] * 32 - c, neg))
    best = jnp.where(key == -1000, 1, 32 - (key & 31))
    o_ref[...] = best.reshape(1, 1, b)


def _tc_vote(cls5, n_points, block=2048):
    nb = n_points // block
    return pl.pallas_call(
        _tc_vote_body,
        grid=(nb,),
        in_specs=[pl.BlockSpec((_SROWS, block), lambda i: (0, i))],
        out_specs=pl.BlockSpec((1, 1, block), lambda i: (i, 0, 0)),
        out_shape=jax.ShapeDtypeStruct((nb, 1, block), jnp.int32),
    )(cls5)


def kernel(proj_range, unproj_range, proj_argmax, px, py):
    h, w = proj_range.shape
    p = unproj_range.shape[0]
    pad = (_S - 1) // 2
    wp = w + 2 * pad
    rng_pad = jnp.pad(proj_range, pad).reshape(-1)
    npix = rng_pad.shape[0]
    cls_pad = jnp.concatenate([
        jnp.pad(proj_argmax, pad).reshape(-1),
        jnp.full((8,), _NCLS, jnp.int32)])
    sentinel = npix
    base = py * wp + px
    offs = [dy * wp + dx for dy in range(_S) for dx in range(_S)]
    idx_all = (jnp.array(offs, jnp.int32)[:, None] + base[None, :])
    g_rng = _sc_gather_range(idx_all.reshape(-1), rng_pad, unproj_range, p)
    g_rng = g_rng.reshape(_KROWS, p)
    w_col = _inv_gauss_weights().reshape(_KROWS, 1)
    offs_col = jnp.array(offs + [0] * (_KROWS - _SS),
                         jnp.int32).reshape(_KROWS, 1)
    base3 = base.reshape(p // 2048, 1, 2048)
    sel_idx = _tc_select(g_rng, base3, w_col, offs_col, sentinel, p)
    cls5 = _sc_gather_cls(sel_idx.reshape(-1), cls_pad, p)
    out3 = _tc_vote(cls5.reshape(_SROWS, p), p)
    return out3.reshape(p)
